# baseline (speedup 1.0000x reference)
import functools

import jax
import jax.numpy as jnp
from jax import lax
from jax.experimental import pallas as pl
from jax.experimental.pallas import tpu as pltpu
from jax.experimental.pallas import tpu_sc as plsc

N_NODES = 10000
D_FEAT = 128
N_EDGES = 320000

_NC = 2
_NS = 16
_NW = _NC * _NS
_LANES = 16

_E_PER_W = N_EDGES // _NW
_B_CH = 80
_N_CH = _E_PER_W // _B_CH
_N_G = _B_CH // _LANES


def _sc_dot_kernel(h_hbm, src_hbm, dst_hbm, out_hbm, hs, tmp, bufs):
    wid = lax.axis_index("s") * _NC + lax.axis_index("c")
    sid = lax.axis_index("s")
    base_w = wid * _E_PER_W

    @pl.when(sid < _NS - 1)
    def _():
        r0 = sid * 624
        pltpu.sync_copy(h_hbm.at[pl.ds(r0, 624)], hs.at[pl.ds(r0, 624)])

    @pl.when(sid == _NS - 1)
    def _():
        pltpu.sync_copy(h_hbm.at[pl.ds(9360, 640)], hs.at[pl.ds(9360, 640)])

    plsc.subcore_barrier()

    def start_idx(ch, slot):
        sidx, didx, srows, drows, obuf, isem, rsem, osem = bufs[slot]
        base = base_w + ch * _B_CH
        pltpu.async_copy(src_hbm.at[pl.ds(base, _B_CH)], sidx, isem)
        pltpu.async_copy(dst_hbm.at[pl.ds(base, _B_CH)], didx, isem)

    def start_rows(ch, slot):
        sidx, didx, srows, drows, obuf, isem, rsem, osem = bufs[slot]
        base = base_w + ch * _B_CH
        pltpu.make_async_copy(src_hbm.at[pl.ds(base, _B_CH)], sidx, isem).wait()
        pltpu.make_async_copy(dst_hbm.at[pl.ds(base, _B_CH)], didx, isem).wait()
        pltpu.async_copy(hs.at[sidx], srows, rsem)
        pltpu.async_copy(hs.at[didx], drows, rsem)

    def wait_rows(slot):
        sidx, didx, srows, drows, obuf, isem, rsem, osem = bufs[slot]
        pltpu.make_async_copy(hs.at[sidx], srows, rsem).wait()
        pltpu.make_async_copy(hs.at[didx], drows, rsem).wait()

    def start_ostore(ch, slot):
        *_, obuf, isem, rsem, osem = bufs[slot]
        base = base_w + ch * _B_CH
        pltpu.async_copy(obuf, out_hbm.at[pl.ds(base, _B_CH)], osem)

    def wait_ostore(ch, slot):
        *_, obuf, isem, rsem, osem = bufs[slot]
        base = base_w + ch * _B_CH
        pltpu.make_async_copy(obuf, out_hbm.at[pl.ds(base, _B_CH)], osem).wait()

    rowid = lax.iota(jnp.int32, _LANES)

    def compute(slot):
        _, _, srows, drows, obuf, _, _, _ = bufs[slot]
        for g in range(_N_G):
            @plsc.parallel_loop(0, _LANES, unroll=4)
            def ebody(e):
                base = g * _LANES + e
                ps = []
                for j in range(D_FEAT // _LANES):
                    sv = srows[base, pl.ds(j * _LANES, _LANES)]
                    dv = drows[base, pl.ds(j * _LANES, _LANES)]
                    ps.append(sv * dv)
                while len(ps) > 1:
                    ps = [ps[i] + ps[i + 1] for i in range(0, len(ps), 2)]
                tmp[e, pl.ds(0, _LANES)] = ps[0]
            cols = [plsc.load_gather(
                        tmp, [rowid, jnp.full((_LANES,), l, jnp.int32)])
                    for l in range(_LANES)]
            while len(cols) > 1:
                cols = [cols[i] + cols[i + 1] for i in range(0, len(cols), 2)]
            obuf[pl.ds(g * _LANES, _LANES)] = cols[0]

    def step(n, slot, *, idx_pref, rows_pref, owait):
        wait_rows(slot)
        if idx_pref:
            start_idx(n + 2, slot)
        if rows_pref:
            start_rows(n + 1, 1 - slot)
        if owait:
            wait_ostore(n - 2, slot)
        compute(slot)
        start_ostore(n, slot)

    start_idx(0, 0)
    start_idx(1, 1)
    start_rows(0, 0)

    step(0, 0, idx_pref=True, rows_pref=True, owait=False)
    step(1, 1, idx_pref=True, rows_pref=True, owait=False)

    def pair_body(i, c):
        n = 2 * i
        step(n, 0, idx_pref=True, rows_pref=True, owait=True)
        step(n + 1, 1, idx_pref=True, rows_pref=True, owait=True)
        return c

    lax.fori_loop(1, 61, pair_body, 0)
    step(122, 0, idx_pref=True, rows_pref=True, owait=True)
    step(123, 1, idx_pref=False, rows_pref=True, owait=True)
    step(124, 0, idx_pref=False, rows_pref=False, owait=True)
    wait_ostore(123, 1)
    wait_ostore(124, 0)


@functools.partial(
    pl.kernel,
    mesh=plsc.VectorSubcoreMesh(core_axis_name="c", subcore_axis_name="s"),
    out_type=jax.ShapeDtypeStruct((N_EDGES,), jnp.float32),
    compiler_params=pltpu.CompilerParams(needs_layout_passes=False),
    scratch_types=[
        pltpu.VMEM_SHARED((N_NODES, D_FEAT), jnp.float32),
        pltpu.VMEM((_LANES, _LANES + 1), jnp.float32),
    ] + [
        t
        for _ in range(2)
        for t in (pltpu.VMEM((_B_CH,), jnp.int32),
                  pltpu.VMEM((_B_CH,), jnp.int32),
                  pltpu.VMEM((_B_CH, D_FEAT), jnp.float32),
                  pltpu.VMEM((_B_CH, D_FEAT), jnp.float32),
                  pltpu.VMEM((_B_CH,), jnp.float32),
                  pltpu.SemaphoreType.DMA,
                  pltpu.SemaphoreType.DMA,
                  pltpu.SemaphoreType.DMA)
    ],
)
def _dot_predictor(h_hbm, src_hbm, dst_hbm, out_hbm, hs, tmp, *flat_bufs):
    bufs = tuple(tuple(flat_bufs[i * 8:(i + 1) * 8]) for i in range(2))
    _sc_dot_kernel(h_hbm, src_hbm, dst_hbm, out_hbm, hs, tmp, bufs)


def kernel(h, edge_index):
    src = edge_index[0]
    dst = edge_index[1]
    return _dot_predictor(h, src, dst)


# speedup vs baseline: 1.2189x; 1.2189x over previous
import functools

import jax
import jax.numpy as jnp
from jax import lax
from jax.experimental import pallas as pl
from jax.experimental.pallas import tpu as pltpu
from jax.experimental.pallas import tpu_sc as plsc

N_NODES = 10000
D_FEAT = 128
N_EDGES = 320000

_NC = 2
_NS = 16
_NW = _NC * _NS
_LANES = 16

_E_PER_W = N_EDGES // _NW
_B_CH = 80
_N_CH = _E_PER_W // _B_CH
_N_G = _B_CH // _LANES


def _sc_dot_kernel(h_hbm, src_hbm, dst_hbm, out_hbm, hs, tmp, bufs):
    wid = lax.axis_index("s") * _NC + lax.axis_index("c")
    sid = lax.axis_index("s")
    base_w = wid * _E_PER_W

    @pl.when(sid < _NS - 1)
    def _():
        r0 = sid * 624
        pltpu.sync_copy(h_hbm.at[pl.ds(r0, 624)], hs.at[pl.ds(r0, 624)])

    @pl.when(sid == _NS - 1)
    def _():
        pltpu.sync_copy(h_hbm.at[pl.ds(9360, 640)], hs.at[pl.ds(9360, 640)])

    plsc.subcore_barrier()

    def start_idx(ch, slot):
        sidx, didx, srows, drows, obuf, isem, rsem, osem = bufs[slot]
        base = base_w + ch * _B_CH
        pltpu.async_copy(src_hbm.at[pl.ds(base, _B_CH)], sidx, isem)
        pltpu.async_copy(dst_hbm.at[pl.ds(base, _B_CH)], didx, isem)

    def start_rows(ch, slot):
        sidx, didx, srows, drows, obuf, isem, rsem, osem = bufs[slot]
        base = base_w + ch * _B_CH
        pltpu.make_async_copy(src_hbm.at[pl.ds(base, _B_CH)], sidx, isem).wait()
        pltpu.make_async_copy(dst_hbm.at[pl.ds(base, _B_CH)], didx, isem).wait()
        pltpu.async_copy(hs.at[sidx], srows, rsem)
        pltpu.async_copy(hs.at[didx], drows, rsem)

    def wait_rows(slot):
        sidx, didx, srows, drows, obuf, isem, rsem, osem = bufs[slot]
        pltpu.make_async_copy(hs.at[sidx], srows, rsem).wait()
        pltpu.make_async_copy(hs.at[didx], drows, rsem).wait()

    def start_ostore(ch, slot):
        *_, obuf, isem, rsem, osem = bufs[slot]
        base = base_w + ch * _B_CH
        pltpu.async_copy(obuf, out_hbm.at[pl.ds(base, _B_CH)], osem)

    def wait_ostore(ch, slot):
        *_, obuf, isem, rsem, osem = bufs[slot]
        base = base_w + ch * _B_CH
        pltpu.make_async_copy(obuf, out_hbm.at[pl.ds(base, _B_CH)], osem).wait()

    rowid = lax.iota(jnp.int32, _LANES)

    def compute(slot):
        _, _, srows, drows, obuf, _, _, _ = bufs[slot]
        for g in range(_N_G):
            @plsc.parallel_loop(0, _LANES, unroll=2)
            def ebody(e):
                base = g * _LANES + e
                a = (srows[base, pl.ds(0, _LANES)] * drows[base, pl.ds(0, _LANES)])
                b = (srows[base, pl.ds(_LANES, _LANES)]
                     * drows[base, pl.ds(_LANES, _LANES)])
                for j in range(2, D_FEAT // _LANES, 2):
                    a = a + (srows[base, pl.ds(j * _LANES, _LANES)]
                             * drows[base, pl.ds(j * _LANES, _LANES)])
                    b = b + (srows[base, pl.ds((j + 1) * _LANES, _LANES)]
                             * drows[base, pl.ds((j + 1) * _LANES, _LANES)])
                tmp[e, pl.ds(0, _LANES)] = a + b
            cols = [plsc.load_gather(
                        tmp, [rowid, jnp.full((_LANES,), l, jnp.int32)])
                    for l in range(_LANES)]
            while len(cols) > 1:
                cols = [cols[i] + cols[i + 1] for i in range(0, len(cols), 2)]
            obuf[pl.ds(g * _LANES, _LANES)] = cols[0]

    def step(n, slot, *, idx_pref, rows_pref, owait):
        wait_rows(slot)
        if idx_pref:
            start_idx(n + 2, slot)
        if rows_pref:
            start_rows(n + 1, 1 - slot)
        if owait:
            wait_ostore(n - 2, slot)
        compute(slot)
        start_ostore(n, slot)

    start_idx(0, 0)
    start_idx(1, 1)
    start_rows(0, 0)

    step(0, 0, idx_pref=True, rows_pref=True, owait=False)
    step(1, 1, idx_pref=True, rows_pref=True, owait=False)

    def pair_body(i, c):
        n = 2 * i
        step(n, 0, idx_pref=True, rows_pref=True, owait=True)
        step(n + 1, 1, idx_pref=True, rows_pref=True, owait=True)
        return c

    lax.fori_loop(1, 61, pair_body, 0)
    step(122, 0, idx_pref=True, rows_pref=True, owait=True)
    step(123, 1, idx_pref=False, rows_pref=True, owait=True)
    step(124, 0, idx_pref=False, rows_pref=False, owait=True)
    wait_ostore(123, 1)
    wait_ostore(124, 0)


@functools.partial(
    pl.kernel,
    mesh=plsc.VectorSubcoreMesh(core_axis_name="c", subcore_axis_name="s"),
    out_type=jax.ShapeDtypeStruct((N_EDGES,), jnp.float32),
    compiler_params=pltpu.CompilerParams(needs_layout_passes=False),
    scratch_types=[
        pltpu.VMEM_SHARED((N_NODES, D_FEAT), jnp.float32),
        pltpu.VMEM((_LANES, _LANES + 1), jnp.float32),
    ] + [
        t
        for _ in range(2)
        for t in (pltpu.VMEM((_B_CH,), jnp.int32),
                  pltpu.VMEM((_B_CH,), jnp.int32),
                  pltpu.VMEM((_B_CH, D_FEAT), jnp.float32),
                  pltpu.VMEM((_B_CH, D_FEAT), jnp.float32),
                  pltpu.VMEM((_B_CH,), jnp.float32),
                  pltpu.SemaphoreType.DMA,
                  pltpu.SemaphoreType.DMA,
                  pltpu.SemaphoreType.DMA)
    ],
)
def _dot_predictor(h_hbm, src_hbm, dst_hbm, out_hbm, hs, tmp, *flat_bufs):
    bufs = tuple(tuple(flat_bufs[i * 8:(i + 1) * 8]) for i in range(2))
    _sc_dot_kernel(h_hbm, src_hbm, dst_hbm, out_hbm, hs, tmp, bufs)


def kernel(h, edge_index):
    src = edge_index[0]
    dst = edge_index[1]
    return _dot_predictor(h, src, dst)


# speedup vs baseline: 1.3337x; 1.0942x over previous
import functools

import jax
import jax.numpy as jnp
from jax import lax
from jax.experimental import pallas as pl
from jax.experimental.pallas import tpu as pltpu
from jax.experimental.pallas import tpu_sc as plsc

N_NODES = 10000
D_FEAT = 128
N_EDGES = 320000

_NC = 2
_NS = 16
_NW = _NC * _NS
_LANES = 16

_E_PER_W = N_EDGES // _NW
_B_CH = 80
_DW = D_FEAT // 2
_N_CH = _E_PER_W // _B_CH
_N_G = _B_CH // _LANES


def _sc_dot_kernel(h_hbm, src_hbm, dst_hbm, out_hbm, hs, tmp, bufs):
    wid = lax.axis_index("s") * _NC + lax.axis_index("c")
    sid = lax.axis_index("s")
    base_w = wid * _E_PER_W

    @pl.when(sid < _NS - 1)
    def _():
        r0 = sid * 624
        pltpu.sync_copy(h_hbm.at[pl.ds(r0, 624)], hs.at[pl.ds(r0, 624)])

    @pl.when(sid == _NS - 1)
    def _():
        pltpu.sync_copy(h_hbm.at[pl.ds(9360, 640)], hs.at[pl.ds(9360, 640)])

    plsc.subcore_barrier()

    def start_idx(ch, slot):
        sidx, didx, srows, drows, obuf, isem, rsem, osem = bufs[slot]
        base = base_w + ch * _B_CH
        pltpu.async_copy(src_hbm.at[pl.ds(base, _B_CH)], sidx, isem)
        pltpu.async_copy(dst_hbm.at[pl.ds(base, _B_CH)], didx, isem)

    def start_rows(ch, slot):
        sidx, didx, srows, drows, obuf, isem, rsem, osem = bufs[slot]
        base = base_w + ch * _B_CH
        pltpu.make_async_copy(src_hbm.at[pl.ds(base, _B_CH)], sidx, isem).wait()
        pltpu.make_async_copy(dst_hbm.at[pl.ds(base, _B_CH)], didx, isem).wait()
        pltpu.async_copy(hs.at[sidx], srows, rsem)
        pltpu.async_copy(hs.at[didx], drows, rsem)

    def wait_rows(slot):
        sidx, didx, srows, drows, obuf, isem, rsem, osem = bufs[slot]
        pltpu.make_async_copy(hs.at[sidx], srows, rsem).wait()
        pltpu.make_async_copy(hs.at[didx], drows, rsem).wait()

    def start_ostore(ch, slot):
        *_, obuf, isem, rsem, osem = bufs[slot]
        base = base_w + ch * _B_CH
        pltpu.async_copy(obuf, out_hbm.at[pl.ds(base, _B_CH)], osem)

    def wait_ostore(ch, slot):
        *_, obuf, isem, rsem, osem = bufs[slot]
        base = base_w + ch * _B_CH
        pltpu.make_async_copy(obuf, out_hbm.at[pl.ds(base, _B_CH)], osem).wait()

    rowid = lax.iota(jnp.int32, _LANES)

    def compute(slot):
        _, _, srows, drows, obuf, _, _, _ = bufs[slot]
        for g in range(_N_G):
            @plsc.parallel_loop(0, _LANES, unroll=2)
            def ebody(e):
                base = g * _LANES + e
                ps = []
                for k in range(_DW // _LANES):
                    si = srows[base, pl.ds(k * _LANES, _LANES)]
                    di = drows[base, pl.ds(k * _LANES, _LANES)]
                    sb = plsc.bitcast(si, jnp.bfloat16)
                    db = plsc.bitcast(di, jnp.bfloat16)
                    p = sb * db
                    lo, hi = plsc.unpack(p, format=plsc.PackFormat.INTERLEAVED)
                    ps.append(lo + hi)
                tmp[e, pl.ds(0, _LANES)] = (ps[0] + ps[1]) + (ps[2] + ps[3])
            cols = [plsc.load_gather(
                        tmp, [rowid, jnp.full((_LANES,), l, jnp.int32)])
                    for l in range(_LANES)]
            while len(cols) > 1:
                cols = [cols[i] + cols[i + 1] for i in range(0, len(cols), 2)]
            obuf[pl.ds(g * _LANES, _LANES)] = cols[0]

    def step(n, slot, *, idx_pref, rows_pref, owait):
        wait_rows(slot)
        if idx_pref:
            start_idx(n + 2, slot)
        if rows_pref:
            start_rows(n + 1, 1 - slot)
        if owait:
            wait_ostore(n - 2, slot)
        compute(slot)
        start_ostore(n, slot)

    start_idx(0, 0)
    start_idx(1, 1)
    start_rows(0, 0)

    step(0, 0, idx_pref=True, rows_pref=True, owait=False)
    step(1, 1, idx_pref=True, rows_pref=True, owait=False)

    def pair_body(i, c):
        n = 2 * i
        step(n, 0, idx_pref=True, rows_pref=True, owait=True)
        step(n + 1, 1, idx_pref=True, rows_pref=True, owait=True)
        return c

    lax.fori_loop(1, 61, pair_body, 0)
    step(122, 0, idx_pref=True, rows_pref=True, owait=True)
    step(123, 1, idx_pref=False, rows_pref=True, owait=True)
    step(124, 0, idx_pref=False, rows_pref=False, owait=True)
    wait_ostore(123, 1)
    wait_ostore(124, 0)


@functools.partial(
    pl.kernel,
    mesh=plsc.VectorSubcoreMesh(core_axis_name="c", subcore_axis_name="s"),
    out_type=jax.ShapeDtypeStruct((N_EDGES,), jnp.float32),
    compiler_params=pltpu.CompilerParams(needs_layout_passes=False),
    scratch_types=[
        pltpu.VMEM_SHARED((N_NODES, _DW), jnp.int32),
        pltpu.VMEM((_LANES, _LANES + 1), jnp.float32),
    ] + [
        t
        for _ in range(2)
        for t in (pltpu.VMEM((_B_CH,), jnp.int32),
                  pltpu.VMEM((_B_CH,), jnp.int32),
                  pltpu.VMEM((_B_CH, _DW), jnp.int32),
                  pltpu.VMEM((_B_CH, _DW), jnp.int32),
                  pltpu.VMEM((_B_CH,), jnp.float32),
                  pltpu.SemaphoreType.DMA,
                  pltpu.SemaphoreType.DMA,
                  pltpu.SemaphoreType.DMA)
    ],
)
def _dot_predictor(h_hbm, src_hbm, dst_hbm, out_hbm, hs, tmp, *flat_bufs):
    bufs = tuple(tuple(flat_bufs[i * 8:(i + 1) * 8]) for i in range(2))
    _sc_dot_kernel(h_hbm, src_hbm, dst_hbm, out_hbm, hs, tmp, bufs)


def kernel(h, edge_index):
    hb = h.astype(jnp.bfloat16)
    hi = jax.lax.bitcast_convert_type(
        hb.reshape(N_NODES, _DW, 2), jnp.int32)
    src = edge_index[0]
    dst = edge_index[1]
    return _dot_predictor(hi, src, dst)
